# baseline (speedup 1.0000x reference)
import functools

import jax
import jax.numpy as jnp
from jax import lax
from jax.experimental import pallas as pl
from jax.experimental.pallas import tpu as pltpu
from jax.experimental.pallas import tpu_sc as plsc

N = 10000
E = 320000
NP = 10240
NSUB = 16
NCORE = 2
EPT = E // NSUB
CHUNK = 112
EPTP = 20160
NCHUNK = EPTP // CHUNK
RPT = NP // NSUB
RB = 80
NRB = RPT // RB
K = 4


def _tc_matmul1(xp, W1s):
    R = 512
    NS, KD, F = W1s.shape

    def body(x_ref, w_ref, o_ref):
        x = x_ref[...]
        for t in range(NS):
            o_ref[t] = jnp.dot(x, w_ref[t], preferred_element_type=jnp.float32)

    return pl.pallas_call(
        body,
        grid=(NP // R,),
        in_specs=[
            pl.BlockSpec((R, xp.shape[1]), lambda i: (i, 0)),
            pl.BlockSpec((NS, KD, F), lambda i: (0, 0, 0)),
        ],
        out_specs=pl.BlockSpec((NS, R, F), lambda i: (0, i, 0)),
        out_shape=jax.ShapeDtypeStruct((NS, NP, F), jnp.float32),
    )(xp, W1s)


def _tc_matmul2(acc1, b1, W2s):
    R = 512
    NS, KD, F = W2s.shape

    def body(a_ref, b_ref, w_ref, o_ref):
        h = jnp.maximum(a_ref[...] + b_ref[0], 0.0)
        for t in range(NS):
            o_ref[t] = jnp.dot(h, w_ref[t], preferred_element_type=jnp.float32)

    return pl.pallas_call(
        body,
        grid=(NP // R,),
        in_specs=[
            pl.BlockSpec((R, acc1.shape[1]), lambda i: (i, 0)),
            pl.BlockSpec((1, b1.shape[1]), lambda i: (0, 0)),
            pl.BlockSpec((NS, KD, F), lambda i: (0, 0, 0)),
        ],
        out_specs=pl.BlockSpec((NS, R, F), lambda i: (0, i, 0)),
        out_shape=jax.ShapeDtypeStruct((NS, NP, F), jnp.float32),
    )(acc1, b1, W2s)


def _fill(ref, nrows, ncols, val):
    row = jnp.full((16,), val, dtype=jnp.float32)

    def body(r, _):
        for k in range(ncols // 16):
            ref[r, pl.ds(16 * k, 16)] = row
        return 0

    lax.fori_loop(0, nrows, body, 0)


def _bcast_gather(tbl, g):
    v = tbl[g // 16]
    idx = jnp.full((16, 1), g % 16, dtype=jnp.int32)
    return lax.gather(
        v, idx,
        dimension_numbers=lax.GatherDimensionNumbers(
            offset_dims=(), collapsed_slice_dims=(0,), start_index_map=(0,)),
        slice_sizes=(1,),
        mode=lax.GatherScatterMode.PROMISE_IN_BOUNDS)


def _make_sc_pass(F, nq, with_hist, with_bias):
    mesh = plsc.VectorSubcoreMesh(core_axis_name="c", subcore_axis_name="s")

    out_type = [jax.ShapeDtypeStruct((NP, 2 * nq * F), jnp.float32)]
    if with_hist:
        out_type += [jax.ShapeDtypeStruct((NSUB, 40, 16), jnp.float32)] * 2

    scratch = [
        pltpu.VMEM((NCHUNK, CHUNK), jnp.int32),
        pltpu.VMEM((NCHUNK, CHUNK), jnp.int32),
        pltpu.VMEM((K, CHUNK, F), jnp.float32),
        pltpu.VMEM((2, RB, F), jnp.float32),
        pltpu.VMEM((RB, F), jnp.float32),
        pltpu.VMEM((40, 16), jnp.float32),
        pltpu.VMEM((40, 16), jnp.float32),
        pltpu.VMEM_SHARED((NP, F), jnp.float32),
        pltpu.VMEM_SHARED((NP, F), jnp.float32),
        pltpu.SemaphoreType.DMA,
        pltpu.SemaphoreType.DMA,
    ]
    if with_hist:
        scratch += [
            pltpu.VMEM((CHUNK, F), jnp.float32),
            pltpu.VMEM((RPT, F), jnp.float32),
        ]
    if with_bias:
        scratch += [pltpu.VMEM((F,), jnp.float32)]

    def body(*refs):
        it = iter(refs)
        xw = next(it)
        adj = next(it)
        if not with_hist:
            binv_in = next(it)
            dinv_in = next(it)
        if with_bias:
            bias_in = next(it)
        out = next(it)
        if with_hist:
            binv_out = next(it)
            dinv_out = next(it)
        node_idx = next(it)
        edge_idx = next(it)
        rows_v = next(it)
        sbuf = next(it)
        zrows = next(it)
        invB = next(it)
        invD = next(it)
        acc_e = next(it)
        acc_o = next(it)
        sem = next(it)
        sem_s = next(it)
        if with_hist:
            onesF = next(it)
            hbuf = next(it)
        if with_bias:
            bias_v = next(it)

        c = lax.axis_index("c")
        s = lax.axis_index("s")
        r0 = s * RPT

        pltpu.sync_copy(adj.at[0, s], node_idx)
        pltpu.sync_copy(adj.at[1, s], edge_idx)

        _fill(zrows, RB, F, 0.0)
        for j in range(NRB):
            pltpu.sync_copy(zrows, acc_e.at[pl.ds(r0 + RB * j, RB)])
            pltpu.sync_copy(zrows, acc_o.at[pl.ds(r0 + RB * j, RB)])
        if with_hist:
            _fill(onesF, CHUNK, F, 1.0)
        plsc.subcore_barrier()

        if with_hist:
            iota16 = lax.iota(jnp.int32, 16)

            with jax.named_scope("hist"):
                def hist_step(i, _):
                    pltpu.async_copy(onesF, acc_o.at[node_idx.at[i]],
                                     sem_s, add=True)
                    pltpu.async_copy(onesF, acc_e.at[edge_idx.at[i]],
                                     sem_s, add=True)
                    return 0

                lax.fori_loop(0, NCHUNK, hist_step, 0)

                def hist_drain(i, _):
                    pltpu.make_async_copy(
                        onesF, acc_o.at[node_idx.at[0]], sem_s).wait()
                    return 0

                lax.fori_loop(0, 2 * NCHUNK, hist_drain, 0)
                plsc.subcore_barrier()

                def compact_inv(acc, dst):
                    pltpu.sync_copy(acc.at[pl.ds(r0, RPT)], hbuf)

                    def body_r(r, _):
                        v = jnp.zeros((16,), jnp.float32)
                        for j in range(16):
                            v = jnp.where(iota16 == j,
                                          hbuf[16 * r + j, pl.ds(0, 16)], v)
                        dst[r] = jnp.where(v > 0.0, 1.0 / v, 0.0)
                        return 0

                    lax.fori_loop(0, 40, body_r, 0)

                compact_inv(acc_o, invD)
                compact_inv(acc_e, invB)
                zds = []
                for j in range(NRB):
                    zds.append(pltpu.async_copy(
                        zrows, acc_e.at[pl.ds(r0 + RB * j, RB)], sem_s))
                    zds.append(pltpu.async_copy(
                        zrows, acc_o.at[pl.ds(r0 + RB * j, RB)], sem_s))
                for d in zds:
                    d.wait()
                plsc.subcore_barrier()

            @pl.when(c == 0)
            def _():
                pltpu.sync_copy(invB, binv_out.at[s])
                pltpu.sync_copy(invD, dinv_out.at[s])
        else:
            pltpu.sync_copy(binv_in.at[s], invB)
            pltpu.sync_copy(dinv_in.at[s], invD)
        if with_bias:
            pltpu.sync_copy(bias_in.at[c], bias_v)

        def pipe_pass(table, acc, sidx, didx):
            def it(g, _):
                ds = []
                for b in range(K):
                    i = g * K + b

                    @pl.when(g > 0)
                    def _():
                        pltpu.make_async_copy(
                            rows_v.at[b], acc.at[didx.at[0]], sem_s).wait()

                    ds.append(pltpu.async_copy(
                        table.at[sidx.at[i]], rows_v.at[b], sem))
                for b in range(K):
                    i = g * K + b
                    ds[b].wait()
                    pltpu.async_copy(rows_v.at[b], acc.at[didx.at[i]],
                                     sem_s, add=True)
                return 0

            lax.fori_loop(0, NCHUNK // K, it, 0)
            for b in range(K):
                pltpu.make_async_copy(
                    rows_v.at[b], acc.at[didx.at[0]], sem_s).wait()

        def scale_phase(acc, inv_tbl, add_bias, store_dst):
            lds = [None] * NRB
            sts = [None] * NRB
            lds[0] = pltpu.async_copy(acc.at[pl.ds(r0, RB)], sbuf.at[0], sem)
            for j in range(NRB):
                if j + 1 < NRB:
                    if j >= 1:
                        sts[j - 1].wait()
                    lds[j + 1] = pltpu.async_copy(
                        acc.at[pl.ds(r0 + RB * (j + 1), RB)],
                        sbuf.at[(j + 1) % 2], sem)
                lds[j].wait()
                buf = sbuf.at[j % 2]

                def body_r(r, _, j=j, buf=buf):
                    bc = _bcast_gather(inv_tbl, RB * j + r)
                    for k in range(F // 16):
                        v = buf[r, pl.ds(16 * k, 16)] * bc
                        if add_bias:
                            v = v + bias_v[pl.ds(16 * k, 16)]
                        buf[r, pl.ds(16 * k, 16)] = v
                    return 0

                lax.fori_loop(0, RB, body_r, 0)
                sts[j] = pltpu.async_copy(buf, store_dst(j), sem_s)
            sts[NRB - 2].wait()
            sts[NRB - 1].wait()

        def quarter(q, _):
            t = 2 * q + c

            @pl.when(q > 0)
            def _():
                zds = []
                for j in range(NRB):
                    zds.append(pltpu.async_copy(
                        zrows, acc_e.at[pl.ds(r0 + RB * j, RB)], sem_s))
                    zds.append(pltpu.async_copy(
                        zrows, acc_o.at[pl.ds(r0 + RB * j, RB)], sem_s))
                for d in zds:
                    d.wait()
                plsc.subcore_barrier()

            with jax.named_scope("p1_gather_scatter"):
                pipe_pass(xw.at[t], acc_e, node_idx, edge_idx)
                plsc.subcore_barrier()

            with jax.named_scope("p2_scale"):
                scale_phase(acc_e, invB, False,
                            lambda j: acc_e.at[pl.ds(r0 + RB * j, RB)])
                plsc.subcore_barrier()

            with jax.named_scope("p3_gather_scatter"):
                pipe_pass(acc_e, acc_o, edge_idx, node_idx)
                plsc.subcore_barrier()

            with jax.named_scope("p4_scale_out"):
                scale_phase(acc_o, invD, with_bias,
                            lambda j: out.at[pl.ds(r0 + RB * j, RB),
                                             pl.ds(F * t, F)])
            return 0

        lax.fori_loop(0, nq, quarter, 0)

    return functools.partial(
        pl.kernel, body, out_type=out_type, mesh=mesh,
        scratch_types=scratch,
        compiler_params=pltpu.CompilerParams(use_tc_tiling_on_sc=False),
    )()


def kernel(x, adj, W1, b1, W2, b2):
    xp = jnp.pad(x, ((0, NP - N), (0, 0)))
    adj_r = jnp.pad(adj.reshape(2, NSUB, EPT), ((0, 0), (0, 0), (0, EPTP - EPT)),
                    constant_values=NP - 1).reshape(2, NSUB, NCHUNK, CHUNK)

    W1s = W1.reshape(W1.shape[0], 4, -1).transpose(1, 0, 2)
    W2s = W2.reshape(W2.shape[0], 2, -1).transpose(1, 0, 2)

    xw = _tc_matmul1(xp, W1s)
    acc1, binv, dinv = _make_sc_pass(32, 2, True, False)(xw, adj_r)
    hw2 = _tc_matmul2(acc1, b1.reshape(1, -1), W2s)
    (out2,) = _make_sc_pass(32, 1, False, True)(
        hw2, adj_r, binv, dinv, b2.reshape(2, -1))
    return out2[:N]


# speedup vs baseline: 1.1779x; 1.1779x over previous
import functools

import jax
import jax.numpy as jnp
from jax import lax
from jax.experimental import pallas as pl
from jax.experimental.pallas import tpu as pltpu
from jax.experimental.pallas import tpu_sc as plsc

N = 10000
E = 320000
NP = 10240
NSUB = 16
NCORE = 2
EPT = E // NSUB
CHUNK = 80
EPTP = 20000
NCHUNK = EPTP // CHUNK
RPT = NP // NSUB
RB = 80
NRB = RPT // RB
K = 5


def _tc_matmul1(xp, W1s):
    R = 512
    NS, KD, F = W1s.shape

    def body(x_ref, w_ref, o_ref):
        x = x_ref[...]
        for t in range(NS):
            o_ref[t] = jnp.dot(x, w_ref[t], preferred_element_type=jnp.float32)

    return pl.pallas_call(
        body,
        grid=(NP // R,),
        in_specs=[
            pl.BlockSpec((R, xp.shape[1]), lambda i: (i, 0)),
            pl.BlockSpec((NS, KD, F), lambda i: (0, 0, 0)),
        ],
        out_specs=pl.BlockSpec((NS, R, F), lambda i: (0, i, 0)),
        out_shape=jax.ShapeDtypeStruct((NS, NP, F), jnp.float32),
    )(xp, W1s)


def _tc_matmul2(acc1, b1, W2s):
    R = 512
    NS, KD, F = W2s.shape

    def body(a_ref, b_ref, w_ref, o_ref):
        h = jnp.maximum(a_ref[...] + b_ref[0], 0.0)
        for t in range(NS):
            o_ref[t] = jnp.dot(h, w_ref[t], preferred_element_type=jnp.float32)

    return pl.pallas_call(
        body,
        grid=(NP // R,),
        in_specs=[
            pl.BlockSpec((R, acc1.shape[1]), lambda i: (i, 0)),
            pl.BlockSpec((1, b1.shape[1]), lambda i: (0, 0)),
            pl.BlockSpec((NS, KD, F), lambda i: (0, 0, 0)),
        ],
        out_specs=pl.BlockSpec((NS, R, F), lambda i: (0, i, 0)),
        out_shape=jax.ShapeDtypeStruct((NS, NP, F), jnp.float32),
    )(acc1, b1, W2s)


def _fill(ref, nrows, ncols, val):
    row = jnp.full((16,), val, dtype=jnp.float32)

    def body(r, _):
        for k in range(ncols // 16):
            ref[r, pl.ds(16 * k, 16)] = row
        return 0

    lax.fori_loop(0, nrows, body, 0)


def _bcast_gather(tbl, g):
    v = tbl[g // 16]
    idx = jnp.full((16, 1), g % 16, dtype=jnp.int32)
    return lax.gather(
        v, idx,
        dimension_numbers=lax.GatherDimensionNumbers(
            offset_dims=(), collapsed_slice_dims=(0,), start_index_map=(0,)),
        slice_sizes=(1,),
        mode=lax.GatherScatterMode.PROMISE_IN_BOUNDS)


def _make_sc_pass(F, nq, with_hist, with_bias):
    mesh = plsc.VectorSubcoreMesh(core_axis_name="c", subcore_axis_name="s")

    out_type = [jax.ShapeDtypeStruct((NP, 2 * nq * F), jnp.float32)]
    if with_hist:
        out_type += [jax.ShapeDtypeStruct((NSUB, 40, 16), jnp.float32)] * 2

    scratch = [
        pltpu.VMEM((NCHUNK, CHUNK), jnp.int32),
        pltpu.VMEM((NCHUNK, CHUNK), jnp.int32),
        pltpu.VMEM((K, CHUNK, F), jnp.float32),
        pltpu.VMEM((2, RB, F), jnp.float32),
        pltpu.VMEM((RB, F), jnp.float32),
        pltpu.VMEM((40, 16), jnp.float32),
        pltpu.VMEM((40, 16), jnp.float32),
        pltpu.VMEM_SHARED((NP, F), jnp.float32),
        pltpu.VMEM_SHARED((NP, F), jnp.float32),
        pltpu.SemaphoreType.DMA,
        pltpu.SemaphoreType.DMA,
    ]
    if with_hist:
        scratch += [
            pltpu.VMEM((CHUNK, 16), jnp.float32),
            pltpu.VMEM((RB, 16), jnp.float32),
            pltpu.VMEM((RPT, 16), jnp.float32),
            pltpu.VMEM_SHARED((NP, 16), jnp.float32),
        ]
    if with_bias:
        scratch += [pltpu.VMEM((F,), jnp.float32)]

    def body(*refs):
        it = iter(refs)
        xw = next(it)
        adj = next(it)
        if not with_hist:
            binv_in = next(it)
            dinv_in = next(it)
        if with_bias:
            bias_in = next(it)
        out = next(it)
        if with_hist:
            binv_out = next(it)
            dinv_out = next(it)
        node_idx = next(it)
        edge_idx = next(it)
        rows_v = next(it)
        sbuf = next(it)
        zrows = next(it)
        invB = next(it)
        invD = next(it)
        acc_e = next(it)
        acc_o = next(it)
        sem = next(it)
        sem_s = next(it)
        if with_hist:
            ones16 = next(it)
            z16 = next(it)
            hbuf = next(it)
            hist = next(it)
        if with_bias:
            bias_v = next(it)

        c = lax.axis_index("c")
        s = lax.axis_index("s")
        r0 = s * RPT

        pltpu.sync_copy(adj.at[0, s], node_idx)
        pltpu.sync_copy(adj.at[1, s], edge_idx)

        _fill(zrows, RB, F, 0.0)
        for j in range(NRB):
            pltpu.sync_copy(zrows, acc_e.at[pl.ds(r0 + RB * j, RB)])
            pltpu.sync_copy(zrows, acc_o.at[pl.ds(r0 + RB * j, RB)])
        if with_hist:
            _fill(ones16, CHUNK, 16, 1.0)
            _fill(z16, RB, 16, 0.0)
            for j in range(NRB):
                pltpu.sync_copy(z16, hist.at[pl.ds(r0 + RB * j, RB)])
        plsc.subcore_barrier()

        if with_hist:
            iota16 = lax.iota(jnp.int32, 16)

            def hist_pass(idx_tbl, dst):
                def hist_step(i, _):
                    pltpu.async_copy(ones16, hist.at[idx_tbl.at[i]],
                                     sem_s, add=True)
                    return 0

                lax.fori_loop(0, NCHUNK, hist_step, 0)

                def hist_drain(i, _):
                    pltpu.make_async_copy(
                        ones16, hist.at[idx_tbl.at[0]], sem_s).wait()
                    return 0

                lax.fori_loop(0, NCHUNK, hist_drain, 0)
                plsc.subcore_barrier()
                pltpu.sync_copy(hist.at[pl.ds(r0, RPT)], hbuf)

                def body_r(r, _):
                    v = jnp.zeros((16,), jnp.float32)
                    for j in range(16):
                        v = jnp.where(iota16 == j, hbuf[16 * r + j], v)
                    dst[r] = jnp.where(v > 0.0, 1.0 / v, 0.0)
                    return 0

                lax.fori_loop(0, 40, body_r, 0)
                plsc.subcore_barrier()

            with jax.named_scope("histD"):
                hist_pass(node_idx, invD)
                for j in range(NRB):
                    pltpu.sync_copy(z16, hist.at[pl.ds(r0 + RB * j, RB)])
                plsc.subcore_barrier()
            with jax.named_scope("histB"):
                hist_pass(edge_idx, invB)

            @pl.when(c == 0)
            def _():
                pltpu.sync_copy(invB, binv_out.at[s])
                pltpu.sync_copy(invD, dinv_out.at[s])
        else:
            pltpu.sync_copy(binv_in.at[s], invB)
            pltpu.sync_copy(dinv_in.at[s], invD)
        if with_bias:
            pltpu.sync_copy(bias_in.at[c], bias_v)

        def pipe_pass(table, acc, sidx, didx):
            def it(g, _):
                ds = []
                for b in range(K):
                    i = g * K + b

                    @pl.when(g > 0)
                    def _():
                        pltpu.make_async_copy(
                            rows_v.at[b], acc.at[didx.at[0]], sem_s).wait()

                    ds.append(pltpu.async_copy(
                        table.at[sidx.at[i]], rows_v.at[b], sem))
                for b in range(K):
                    i = g * K + b
                    ds[b].wait()
                    pltpu.async_copy(rows_v.at[b], acc.at[didx.at[i]],
                                     sem_s, add=True)
                return 0

            lax.fori_loop(0, NCHUNK // K, it, 0)
            for b in range(K):
                pltpu.make_async_copy(
                    rows_v.at[b], acc.at[didx.at[0]], sem_s).wait()

        def scale_phase(acc, inv_tbl, add_bias, store_dst):
            lds = [None] * NRB
            sts = [None] * NRB
            lds[0] = pltpu.async_copy(acc.at[pl.ds(r0, RB)], sbuf.at[0], sem)
            for j in range(NRB):
                if j + 1 < NRB:
                    if j >= 1:
                        sts[j - 1].wait()
                    lds[j + 1] = pltpu.async_copy(
                        acc.at[pl.ds(r0 + RB * (j + 1), RB)],
                        sbuf.at[(j + 1) % 2], sem)
                lds[j].wait()
                buf = sbuf.at[j % 2]

                def body_r(r, _, j=j, buf=buf):
                    bc = _bcast_gather(inv_tbl, RB * j + r)
                    for k in range(F // 16):
                        v = buf[r, pl.ds(16 * k, 16)] * bc
                        if add_bias:
                            v = v + bias_v[pl.ds(16 * k, 16)]
                        buf[r, pl.ds(16 * k, 16)] = v
                    return 0

                lax.fori_loop(0, RB, body_r, 0)
                sts[j] = pltpu.async_copy(buf, store_dst(j), sem_s)
            sts[NRB - 2].wait()
            sts[NRB - 1].wait()

        def quarter(q, _):
            t = 2 * q + c

            @pl.when(q > 0)
            def _():
                zds = []
                for j in range(NRB):
                    zds.append(pltpu.async_copy(
                        zrows, acc_e.at[pl.ds(r0 + RB * j, RB)], sem_s))
                    zds.append(pltpu.async_copy(
                        zrows, acc_o.at[pl.ds(r0 + RB * j, RB)], sem_s))
                for d in zds:
                    d.wait()
                plsc.subcore_barrier()

            with jax.named_scope("p1_gather_scatter"):
                pipe_pass(xw.at[t], acc_e, node_idx, edge_idx)
                plsc.subcore_barrier()

            with jax.named_scope("p2_scale"):
                scale_phase(acc_e, invB, False,
                            lambda j: acc_e.at[pl.ds(r0 + RB * j, RB)])
                plsc.subcore_barrier()

            with jax.named_scope("p3_gather_scatter"):
                pipe_pass(acc_e, acc_o, edge_idx, node_idx)
                plsc.subcore_barrier()

            with jax.named_scope("p4_scale_out"):
                scale_phase(acc_o, invD, with_bias,
                            lambda j: out.at[pl.ds(r0 + RB * j, RB),
                                             pl.ds(F * t, F)])
            return 0

        lax.fori_loop(0, nq, quarter, 0)

    return functools.partial(
        pl.kernel, body, out_type=out_type, mesh=mesh,
        scratch_types=scratch,
        compiler_params=pltpu.CompilerParams(use_tc_tiling_on_sc=False),
    )()


def kernel(x, adj, W1, b1, W2, b2):
    xp = jnp.pad(x, ((0, NP - N), (0, 0)))
    adj_r = jnp.pad(adj.reshape(2, NSUB, EPT), ((0, 0), (0, 0), (0, EPTP - EPT)),
                    constant_values=NP - 1).reshape(2, NSUB, NCHUNK, CHUNK)

    W1s = W1.reshape(W1.shape[0], 4, -1).transpose(1, 0, 2)
    W2s = W2.reshape(W2.shape[0], 2, -1).transpose(1, 0, 2)

    xw = _tc_matmul1(xp, W1s)
    acc1, binv, dinv = _make_sc_pass(32, 2, True, False)(xw, adj_r)
    hw2 = _tc_matmul2(acc1, b1.reshape(1, -1), W2s)
    (out2,) = _make_sc_pass(32, 1, False, True)(
        hw2, adj_r, binv, dinv, b2.reshape(2, -1))
    return out2[:N]
